# Initial kernel scaffold; baseline (speedup 1.0000x reference)
import functools

import jax
import jax.numpy as jnp
from jax import lax
from jax.experimental import pallas as pl
from jax.experimental.pallas import tpu as pltpu
from jax.experimental.pallas import tpu_sc as plsc

VOCAB = 49408
MAX_LEN = 77
DIM = 768
BATCH = 1024
SEQ = 77
N = BATCH * SEQ
NW = 32
PER_W = N // NW
CHUNK = 32
NCH = PER_W // CHUNK
LANES = 16
SEG = DIM // LANES


_mesh = plsc.VectorSubcoreMesh(core_axis_name="c", subcore_axis_name="s")


@functools.partial(
    pl.kernel,
    mesh=_mesh,
    out_type=jax.ShapeDtypeStruct((N, DIM), jnp.float32),
    scratch_types=[
        pltpu.VMEM((NCH, CHUNK), jnp.int32),
        pltpu.VMEM((NCH, CHUNK), jnp.int32),
        pltpu.VMEM((CHUNK, DIM), jnp.float32),
        pltpu.VMEM((CHUNK, DIM), jnp.float32),
        pltpu.SemaphoreType.DMA,
        pltpu.SemaphoreType.DMA,
    ],
)
def _emb(tok_idx, pos_idx, tok_tab, pos_tab, out, idx_t, idx_p,
         tok_buf, pos_buf, sem_t, sem_p):
    wid = lax.axis_index("s") * 2 + lax.axis_index("c")
    pltpu.sync_copy(tok_idx.at[pl.ds(wid * NCH, NCH)], idx_t)
    pltpu.sync_copy(pos_idx.at[pl.ds(wid * NCH, NCH)], idx_p)

    def chunk_body(i, carry):
        ct = pltpu.async_copy(tok_tab.at[idx_t.at[i]], tok_buf, sem_t)
        cp = pltpu.async_copy(pos_tab.at[idx_p.at[i]], pos_buf, sem_p)
        ct.wait()
        cp.wait()

        def row_body(r, carry2):
            for j in range(SEG):
                v = pos_buf[r, pl.ds(j * LANES, LANES)]
                plsc.addupdate(tok_buf.at[r, pl.ds(j * LANES, LANES)], v)
            return carry2

        lax.fori_loop(0, CHUNK, row_body, 0)
        pltpu.sync_copy(tok_buf, out.at[pl.ds((wid * NCH + i) * CHUNK, CHUNK)])
        return carry

    lax.fori_loop(0, NCH, chunk_body, 0)


def kernel(tokens, positions, token_table, pos_table):
    tok = tokens.reshape(N // CHUNK, CHUNK).astype(jnp.int32)
    pos = positions.reshape(N // CHUNK, CHUNK).astype(jnp.int32)
    out = _emb(tok, pos, token_table, pos_table)
    return out.reshape(BATCH, SEQ, DIM)


# speedup vs baseline: 1.4589x; 1.4589x over previous
import functools

import jax
import jax.numpy as jnp
from jax import lax
from jax.experimental import pallas as pl
from jax.experimental.pallas import tpu as pltpu
from jax.experimental.pallas import tpu_sc as plsc

VOCAB = 49408
MAX_LEN = 77
DIM = 768
BATCH = 1024
SEQ = 77
N = BATCH * SEQ
NW = 32
PER_W = N // NW
CHUNK = 32
NCH = PER_W // CHUNK
LANES = 16
SEG = DIM // LANES


_mesh = plsc.VectorSubcoreMesh(core_axis_name="c", subcore_axis_name="s")


@functools.partial(
    pl.kernel,
    mesh=_mesh,
    out_type=jax.ShapeDtypeStruct((N, DIM), jnp.float32),
    scratch_types=[
        pltpu.VMEM((NCH, CHUNK), jnp.int32),
        pltpu.VMEM((NCH, CHUNK), jnp.int32),
        pltpu.VMEM((CHUNK, DIM), jnp.float32),
        pltpu.VMEM((CHUNK, DIM), jnp.float32),
        pltpu.SemaphoreType.DMA,
        pltpu.SemaphoreType.DMA,
    ],
)
def _emb(tok_idx, pos_idx, tok_tab, pos_tab, out, idx_t, idx_p,
         tok_buf, pos_buf, sem_t, sem_p):
    wid = lax.axis_index("s") * 2 + lax.axis_index("c")
    pltpu.sync_copy(tok_idx.at[wid], idx_t)
    pltpu.sync_copy(pos_idx.at[wid], idx_p)

    def chunk_body(i, carry):
        ct = pltpu.async_copy(tok_tab.at[idx_t.at[i]], tok_buf, sem_t)
        cp = pltpu.async_copy(pos_tab.at[idx_p.at[i]], pos_buf, sem_p)
        ct.wait()
        cp.wait()

        def row_body(r, carry2):
            for j in range(SEG):
                v = pos_buf[r, pl.ds(j * LANES, LANES)]
                plsc.addupdate(tok_buf.at[r, pl.ds(j * LANES, LANES)], v)
            return carry2

        lax.fori_loop(0, CHUNK, row_body, 0)
        pltpu.sync_copy(tok_buf, out.at[pl.ds((wid * NCH + i) * CHUNK, CHUNK)])
        return carry

    lax.fori_loop(0, NCH, chunk_body, 0)


def kernel(tokens, positions, token_table, pos_table):
    tok = tokens.reshape(NW, NCH, CHUNK).astype(jnp.int32)
    pos = positions.reshape(NW, NCH, CHUNK).astype(jnp.int32)
    out = _emb(tok, pos, token_table, pos_table)
    return out.reshape(BATCH, SEQ, DIM)
